# baseline (speedup 1.0000x reference)
import jax
import jax.numpy as jnp
from jax.experimental import pallas as pl
from jax.experimental.pallas import tpu as pltpu

_N = 262144
_D = 64
_C = 10
_T = 4096

_S_A = 8
_IMG_ROWS = _N * 3 // 128
_SEG_ROWS = _N // 64
_IMG_BLK = _IMG_ROWS // (2 * _S_A)
_SEG_BLK = _SEG_ROWS // (2 * _S_A)


def _msece_body(img_a_ref, img_b_ref, gts_ref, seg_ref, g_ref, out_ref):
    s = pl.program_id(1)

    @pl.when(s == 0)
    def _init():
        out_ref[...] = jnp.zeros_like(out_ref)

    d = img_a_ref[...] - img_b_ref[...]
    mse_part = jnp.sum(d * d, axis=0, keepdims=True)

    seg = seg_ref[...]
    dot640 = jnp.sum(gts_ref[...] * seg, axis=0, keepdims=True)
    dot_part = (dot640[:, 0:128] + dot640[:, 128:256] + dot640[:, 256:384]
                + dot640[:, 384:512] + dot640[:, 512:640])

    z = jnp.exp(seg)
    gs = jnp.dot(z, g_ref[...], preferred_element_type=jnp.float32)
    lse_part = jnp.sum(jnp.log(gs), axis=0, keepdims=True)

    out_ref[0, 0:1, :] += mse_part
    out_ref[0, 1:2, :] += dot_part
    out_ref[0, 2:3, 0:64] += lse_part


def _msece_partials(img_a, img_b, gts, seg, g):
    grid = (2, _S_A)
    return pl.pallas_call(
        _msece_body,
        grid=grid,
        in_specs=[
            pl.BlockSpec((_IMG_BLK, 128), lambda c, s: (c * _S_A + s, 0)),
            pl.BlockSpec((_IMG_BLK, 128), lambda c, s: (c * _S_A + s, 0)),
            pl.BlockSpec((_SEG_BLK, 640), lambda c, s: (c * _S_A + s, 0)),
            pl.BlockSpec((_SEG_BLK, 640), lambda c, s: (c * _S_A + s, 0)),
            pl.BlockSpec((640, 64), lambda c, s: (0, 0)),
        ],
        out_specs=pl.BlockSpec((1, 3, 128), lambda c, s: (c, 0, 0)),
        out_shape=jax.ShapeDtypeStruct((2, 3, 128), jnp.float32),
        compiler_params=pltpu.CompilerParams(
            dimension_semantics=("parallel", "arbitrary"),
        ),
    )(img_a, img_b, gts, seg, g)


_G_TRIP = 8
_NB = _T // (2 * _G_TRIP)


def _triplet_body(a_sref, p_sref, n_sref, feats, out_ref, gbuf, sems):
    c = pl.program_id(0)
    base = c * (_T // 2)
    srefs = (a_sref, p_sref, n_sref)

    def issue(b):
        slot = jax.lax.rem(b, 2)
        tb = base + b * _G_TRIP
        for w in range(3):
            for j in range(_G_TRIP):
                idx = srefs[w][tb + j]
                r0 = pl.multiple_of((idx >> 3) << 3, 8)
                pltpu.make_async_copy(
                    feats.at[0, pl.ds(r0, 8), :],
                    gbuf.at[slot, w * _G_TRIP + j],
                    sems.at[slot],
                ).start()

    def wait(b):
        slot = jax.lax.rem(b, 2)
        pltpu.make_async_copy(gbuf.at[slot], gbuf.at[slot], sems.at[slot]).wait()

    def process(b):
        slot = jax.lax.rem(b, 2)
        tb = base + b * _G_TRIP
        rows = []
        for w in range(3):
            rws = []
            for j in range(_G_TRIP):
                idx = srefs[w][tb + j]
                chunk = gbuf[slot, w * _G_TRIP + j]
                rws.append(pltpu.roll(chunk, -(idx & 7), axis=0)[0:1, :])
            rows.append(jnp.concatenate(rws, axis=0))
        a, p, n = rows
        dp = a - p
        dn = a - n
        dp2 = jnp.sum(dp * dp, axis=1, keepdims=True)
        dn2 = jnp.sum(dn * dn, axis=1, keepdims=True)
        return jnp.maximum(jnp.sqrt(dp2) - jnp.sqrt(dn2), 0.0)

    issue(0)
    issue(1)

    def body(b, acc):
        wait(b)
        contrib = process(b)

        @pl.when(b < _NB - 2)
        def _():
            issue(b + 2)

        return acc + contrib

    acc = jax.lax.fori_loop(0, _NB, body, jnp.zeros((_G_TRIP, 1), jnp.float32))
    out_ref[0] = acc


def _triplet_partials(feats, anchor_idx, pos_idx, neg_idx):
    grid_spec = pltpu.PrefetchScalarGridSpec(
        num_scalar_prefetch=3,
        grid=(2,),
        in_specs=[pl.BlockSpec(memory_space=pl.ANY)],
        out_specs=pl.BlockSpec((1, _G_TRIP, 1), lambda c, a, p, n: (c, 0, 0)),
        scratch_shapes=[
            pltpu.VMEM((2, 3 * _G_TRIP, 8, _D), jnp.float32),
            pltpu.SemaphoreType.DMA((2,)),
        ],
    )
    return pl.pallas_call(
        _triplet_body,
        grid_spec=grid_spec,
        out_shape=jax.ShapeDtypeStruct((2, _G_TRIP, 1), jnp.float32),
        compiler_params=pltpu.CompilerParams(
            dimension_semantics=("parallel",),
        ),
    )(anchor_idx, pos_idx, neg_idx, feats)


def kernel(gt_img, gt_seg, inr_output, seg_output, inr_features,
           anchor_idx, pos_idx, neg_idx):
    img_a = gt_img.reshape(_IMG_ROWS, 128)
    img_b = inr_output.reshape(_IMG_ROWS, 128)
    gts = gt_seg.reshape(_SEG_ROWS, 640)
    seg = seg_output.reshape(_SEG_ROWS, 640)
    g = (jnp.arange(640, dtype=jnp.int32)[:, None] // _C
         == jnp.arange(64, dtype=jnp.int32)[None, :]).astype(jnp.float32)

    out_a = _msece_partials(img_a, img_b, gts, seg, g)
    out_b = _triplet_partials(inr_features, anchor_idx, pos_idx, neg_idx)

    mse = jnp.sum(out_a[:, 0, :]) / (_N * 3)
    ce = (jnp.sum(out_a[:, 2, :]) - jnp.sum(out_a[:, 1, :])) / _N
    triplet = jnp.sum(out_b)
    return mse + ce + triplet


# speedup vs baseline: 2.6499x; 2.6499x over previous
import jax
import jax.numpy as jnp
from jax.experimental import pallas as pl
from jax.experimental.pallas import tpu as pltpu

_N = 262144
_D = 64
_C = 10
_T = 4096

_S_A = 8
_IMG_ROWS = _N * 3 // 128
_SEG_COLS = _N // 128
_IMG_BLK = _IMG_ROWS // (2 * _S_A)
_SEG_BLK = _SEG_COLS // (2 * _S_A)


def _msece_body(img_a_ref, img_b_ref, gts_ref, seg_ref, out_ref):
    s = pl.program_id(1)

    @pl.when(s == 0)
    def _init():
        out_ref[...] = jnp.zeros_like(out_ref)

    d = img_a_ref[...] - img_b_ref[...]
    mse_part = jnp.sum(d * d, axis=0, keepdims=True)

    seg = seg_ref[...]
    gts = gts_ref[...]
    z = jnp.exp(seg)
    lse = jnp.log(jnp.sum(z, axis=0))
    lse_part = jnp.sum(lse, axis=0, keepdims=True)
    dot_part = jnp.sum(gts * seg, axis=(0, 1))[None, :]

    out_ref[0, 0:1, :] += mse_part
    out_ref[0, 1:2, :] += dot_part
    out_ref[0, 2:3, :] += lse_part


def _msece_partials(img_a, img_b, gts, seg):
    grid = (2, _S_A)
    return pl.pallas_call(
        _msece_body,
        grid=grid,
        in_specs=[
            pl.BlockSpec((_IMG_BLK, 128), lambda c, s: (c * _S_A + s, 0)),
            pl.BlockSpec((_IMG_BLK, 128), lambda c, s: (c * _S_A + s, 0)),
            pl.BlockSpec((_C, _SEG_BLK, 128), lambda c, s: (0, c * _S_A + s, 0)),
            pl.BlockSpec((_C, _SEG_BLK, 128), lambda c, s: (0, c * _S_A + s, 0)),
        ],
        out_specs=pl.BlockSpec((1, 3, 128), lambda c, s: (c, 0, 0)),
        out_shape=jax.ShapeDtypeStruct((2, 3, 128), jnp.float32),
        compiler_params=pltpu.CompilerParams(
            dimension_semantics=("parallel", "arbitrary"),
        ),
    )(img_a, img_b, gts, seg)


_G_TRIP = 8
_NB = _T // (2 * _G_TRIP)


def _triplet_body(a_sref, p_sref, n_sref, feats, out_ref, gbuf, sems):
    c = pl.program_id(0)
    base = c * (_T // 2)
    srefs = (a_sref, p_sref, n_sref)

    def issue(b):
        slot = jax.lax.rem(b, 2)
        tb = base + b * _G_TRIP
        for w in range(3):
            for j in range(_G_TRIP):
                idx = srefs[w][tb + j]
                r0 = pl.multiple_of((idx >> 3) << 3, 8)
                pltpu.make_async_copy(
                    feats.at[pl.ds(r0, 8), :],
                    gbuf.at[slot, w * _G_TRIP + j],
                    sems.at[slot],
                ).start()

    def wait(b):
        slot = jax.lax.rem(b, 2)
        pltpu.make_async_copy(gbuf.at[slot], gbuf.at[slot], sems.at[slot]).wait()

    def process(b):
        slot = jax.lax.rem(b, 2)
        tb = base + b * _G_TRIP
        rows = []
        for w in range(3):
            rws = []
            for j in range(_G_TRIP):
                idx = srefs[w][tb + j]
                chunk = gbuf[slot, w * _G_TRIP + j]
                rws.append(pltpu.roll(chunk, -(idx & 7), axis=0)[0:1, :])
            rows.append(jnp.concatenate(rws, axis=0))
        a, p, n = rows
        dp = a - p
        dn = a - n
        dp2 = jnp.sum(dp * dp, axis=1, keepdims=True)
        dn2 = jnp.sum(dn * dn, axis=1, keepdims=True)
        return jnp.maximum(jnp.sqrt(dp2) - jnp.sqrt(dn2), 0.0)

    issue(0)
    issue(1)

    def body(b, acc):
        wait(b)
        contrib = process(b)

        @pl.when(b < _NB - 2)
        def _():
            issue(b + 2)

        return acc + contrib

    acc = jax.lax.fori_loop(0, _NB, body, jnp.zeros((_G_TRIP, 1), jnp.float32))
    out_ref[0] = acc


def _triplet_partials(feats, anchor_idx, pos_idx, neg_idx):
    grid_spec = pltpu.PrefetchScalarGridSpec(
        num_scalar_prefetch=3,
        grid=(2,),
        in_specs=[pl.BlockSpec(memory_space=pl.ANY)],
        out_specs=pl.BlockSpec((1, _G_TRIP, 1), lambda c, a, p, n: (c, 0, 0)),
        scratch_shapes=[
            pltpu.VMEM((2, 3 * _G_TRIP, 8, _D), jnp.float32),
            pltpu.SemaphoreType.DMA((2,)),
        ],
    )
    return pl.pallas_call(
        _triplet_body,
        grid_spec=grid_spec,
        out_shape=jax.ShapeDtypeStruct((2, _G_TRIP, 1), jnp.float32),
        compiler_params=pltpu.CompilerParams(
            dimension_semantics=("parallel",),
            disable_bounds_checks=True,
        ),
    )(anchor_idx, pos_idx, neg_idx, feats)


def kernel(gt_img, gt_seg, inr_output, seg_output, inr_features,
           anchor_idx, pos_idx, neg_idx):
    img_a = jnp.transpose(gt_img, (2, 0, 1)).reshape(_IMG_ROWS, 128)
    img_b = jnp.transpose(inr_output, (2, 0, 1)).reshape(_IMG_ROWS, 128)
    gts = jnp.transpose(gt_seg, (2, 0, 1)).reshape(_C, _SEG_COLS, 128)
    seg = jnp.transpose(seg_output, (2, 0, 1)).reshape(_C, _SEG_COLS, 128)

    out_a = _msece_partials(img_a, img_b, gts, seg)
    feats_rm = inr_features.reshape(_N, _D)
    out_b = _triplet_partials(feats_rm, anchor_idx, pos_idx, neg_idx)

    mse = jnp.sum(out_a[:, 0, :]) / (_N * 3)
    ce = (jnp.sum(out_a[:, 2, :]) - jnp.sum(out_a[:, 1, :])) / _N
    triplet = jnp.sum(out_b)
    return mse + ce + triplet


# speedup vs baseline: 6.4953x; 2.4511x over previous
import jax
import jax.numpy as jnp
from jax.experimental import pallas as pl
from jax.experimental.pallas import tpu as pltpu

_N = 262144
_D = 64
_C = 10
_T = 4096

_S_A = 8
_IMG_ROWS = _N * 3 // 128
_SEG_COLS = _N // 128
_IMG_BLK = _IMG_ROWS // (2 * _S_A)
_SEG_BLK = _SEG_COLS // (2 * _S_A)


def _msece_body(img_a_ref, img_b_ref, gts_ref, seg_ref, out_ref):
    s = pl.program_id(1)

    @pl.when(s == 0)
    def _init():
        out_ref[...] = jnp.zeros_like(out_ref)

    d = img_a_ref[...] - img_b_ref[...]
    mse_part = jnp.sum(d * d, axis=0, keepdims=True)

    seg = seg_ref[...]
    gts = gts_ref[...]
    z = jnp.exp(seg)
    lse = jnp.log(jnp.sum(z, axis=0))
    lse_part = jnp.sum(lse, axis=0, keepdims=True)
    dot_part = jnp.sum(gts * seg, axis=(0, 1))[None, :]

    out_ref[0, 0:1, :] += mse_part
    out_ref[0, 1:2, :] += dot_part
    out_ref[0, 2:3, :] += lse_part


def _msece_partials(img_a, img_b, gts, seg):
    grid = (2, _S_A)
    return pl.pallas_call(
        _msece_body,
        grid=grid,
        in_specs=[
            pl.BlockSpec((_IMG_BLK, 128), lambda c, s: (c * _S_A + s, 0)),
            pl.BlockSpec((_IMG_BLK, 128), lambda c, s: (c * _S_A + s, 0)),
            pl.BlockSpec((_C, _SEG_BLK, 128), lambda c, s: (0, c * _S_A + s, 0)),
            pl.BlockSpec((_C, _SEG_BLK, 128), lambda c, s: (0, c * _S_A + s, 0)),
        ],
        out_specs=pl.BlockSpec((1, 3, 128), lambda c, s: (c, 0, 0)),
        out_shape=jax.ShapeDtypeStruct((2, 3, 128), jnp.float32),
        compiler_params=pltpu.CompilerParams(
            dimension_semantics=("parallel", "arbitrary"),
        ),
    )(img_a, img_b, gts, seg)


_K_TRIP = 32
_NB = _T // (2 * _K_TRIP)
_NSLOT = 4


def _triplet_body(a_sref, p_sref, n_sref, feats, out_ref, gbuf, sems):
    c = pl.program_id(0)
    base = c * (_T // 2)
    srefs = (a_sref, p_sref, n_sref)

    def issue(b):
        slot = b & (_NSLOT - 1)
        tb = base + b * _K_TRIP
        for w in range(3):
            for j in range(_K_TRIP):
                idx = srefs[w][tb + j]
                r0 = pl.multiple_of((idx >> 3) << 3, 8)
                pltpu.make_async_copy(
                    feats.at[pl.ds(r0, 8), :],
                    gbuf.at[slot, w * _K_TRIP + j],
                    sems.at[slot],
                ).start()

    def wait(b):
        slot = b & (_NSLOT - 1)
        pltpu.make_async_copy(gbuf.at[slot], gbuf.at[slot], sems.at[slot]).wait()

    def process(b, acc):
        slot = b & (_NSLOT - 1)
        tb = base + b * _K_TRIP
        for g in range(_K_TRIP // 8):
            rows = []
            for w in range(3):
                rws = []
                for j in range(g * 8, g * 8 + 8):
                    idx = srefs[w][tb + j]
                    chunk = gbuf[slot, w * _K_TRIP + j]
                    rws.append(pltpu.roll(chunk, -(idx & 7), axis=0)[0:1, :])
                rows.append(jnp.concatenate(rws, axis=0))
            a, p, n = rows
            dp = a - p
            dn = a - n
            dp2 = jnp.sum(dp * dp, axis=1, keepdims=True)
            dn2 = jnp.sum(dn * dn, axis=1, keepdims=True)
            acc = acc + jnp.maximum(jnp.sqrt(dp2) - jnp.sqrt(dn2), 0.0)
        return acc

    for b in range(_NSLOT - 1):
        issue(b)

    def body(b, acc):
        wait(b)
        acc = process(b, acc)

        @pl.when(b < _NB - (_NSLOT - 1))
        def _():
            issue(b + (_NSLOT - 1))

        return acc

    acc = jax.lax.fori_loop(0, _NB, body, jnp.zeros((8, 1), jnp.float32))
    out_ref[0] = acc


def _triplet_partials(feats, anchor_idx, pos_idx, neg_idx):
    grid_spec = pltpu.PrefetchScalarGridSpec(
        num_scalar_prefetch=3,
        grid=(2,),
        in_specs=[pl.BlockSpec(memory_space=pl.ANY)],
        out_specs=pl.BlockSpec((1, 8, 1), lambda c, a, p, n: (c, 0, 0)),
        scratch_shapes=[
            pltpu.VMEM((_NSLOT, 3 * _K_TRIP, 8, _D), jnp.float32),
            pltpu.SemaphoreType.DMA((_NSLOT,)),
        ],
    )
    return pl.pallas_call(
        _triplet_body,
        grid_spec=grid_spec,
        out_shape=jax.ShapeDtypeStruct((2, 8, 1), jnp.float32),
        compiler_params=pltpu.CompilerParams(
            dimension_semantics=("parallel",),
            disable_bounds_checks=True,
        ),
    )(anchor_idx, pos_idx, neg_idx, feats)


def kernel(gt_img, gt_seg, inr_output, seg_output, inr_features,
           anchor_idx, pos_idx, neg_idx):
    img_a = jnp.transpose(gt_img, (2, 0, 1)).reshape(_IMG_ROWS, 128)
    img_b = jnp.transpose(inr_output, (2, 0, 1)).reshape(_IMG_ROWS, 128)
    gts = jnp.transpose(gt_seg, (2, 0, 1)).reshape(_C, _SEG_COLS, 128)
    seg = jnp.transpose(seg_output, (2, 0, 1)).reshape(_C, _SEG_COLS, 128)

    out_a = _msece_partials(img_a, img_b, gts, seg)
    feats_rm = inr_features.reshape(_N, _D)
    out_b = _triplet_partials(feats_rm, anchor_idx, pos_idx, neg_idx)

    mse = jnp.sum(out_a[:, 0, :]) / (_N * 3)
    ce = (jnp.sum(out_a[:, 2, :]) - jnp.sum(out_a[:, 1, :])) / _N
    triplet = jnp.sum(out_b)
    return mse + ce + triplet
